# baseline (speedup 1.0000x reference)
import dataclasses

import jax
import jax.numpy as jnp
from jax import lax
from jax.experimental import pallas as pl
from jax.experimental.pallas import tpu as pltpu
from jax.experimental.pallas import tpu_sc as plsc

K_TOP = 2
BLK = 256
NBLK = 24
NC, NS = 2, 16
NW = NC * NS


def _router_meta_kernel(x_ref, gw_ref, p0_ref, p1_ref,
                        w0_ref, w1_ref, be_ref):
    x = x_ref[...]
    T = x.shape[0]
    logits = jnp.dot(x, gw_ref[...].T,
                     preferred_element_type=jnp.float32)
    m = jnp.max(logits, axis=-1, keepdims=True)
    ex = jnp.exp(logits - m)
    p = ex / jnp.sum(ex, axis=-1, keepdims=True)
    num_experts = p.shape[-1]
    idx = lax.broadcasted_iota(jnp.int32, p.shape, 1)
    m1 = jnp.max(p, axis=-1, keepdims=True)
    i1 = jnp.min(jnp.where(p == m1, idx, num_experts), axis=-1, keepdims=True)
    is1 = idx == i1
    p2 = jnp.where(is1, -jnp.inf, p)
    m2 = jnp.max(p2, axis=-1, keepdims=True)
    i2 = jnp.min(jnp.where(p2 == m2, idx, num_experts), axis=-1, keepdims=True)
    is2 = idx == i2
    sel = is1 | is2
    wsum = m1 + m2
    w0_ref[...] = m1 / wsum
    w1_ref[...] = m2 / wsum

    selb = sel.astype(jnp.bfloat16)
    r_io = lax.broadcasted_iota(jnp.int32, (T, T), 0)
    c_io = lax.broadcasted_iota(jnp.int32, (T, T), 1)
    ltri = (c_io < r_io).astype(jnp.bfloat16)
    rank = jnp.dot(ltri, selb, preferred_element_type=jnp.float32)

    cnt = jnp.sum(sel.astype(jnp.float32), axis=0, keepdims=True)
    nblk = jnp.floor((cnt + (BLK - 1)) * (1.0 / BLK))
    e_r = lax.broadcasted_iota(jnp.int32, (num_experts, num_experts), 0)
    e_c = lax.broadcasted_iota(jnp.int32, (num_experts, num_experts), 1)
    excl = (e_r < e_c).astype(jnp.float32)
    blk_off = jnp.dot(nblk, excl,
                      preferred_element_type=jnp.float32)
    pad_off = blk_off * BLK
    pos = pad_off + rank
    p0_ref[...] = jnp.sum(jnp.where(is1, pos, 0.0), axis=1,
                          keepdims=True).astype(jnp.int32)
    p1_ref[...] = jnp.sum(jnp.where(is2, pos, 0.0), axis=1,
                          keepdims=True).astype(jnp.int32)

    blk_cum = blk_off + nblk
    lane8 = lax.broadcasted_iota(jnp.int32, (1, num_experts), 1)
    s_io = lax.broadcasted_iota(jnp.int32, (1, NBLK), 1).astype(jnp.float32)
    be = jnp.zeros((1, NBLK), jnp.int32)
    for e in range(num_experts):
        ce = jnp.sum(jnp.where(lane8 == e, blk_cum, 0.0),
                     axis=1, keepdims=True)
        be = be + (s_io >= ce).astype(jnp.int32)
    be_ref[...] = jnp.minimum(be, num_experts - 1)


def _cast_kernel(wg_ref, wu_ref, wd_ref, og_ref, ou_ref, od_ref):
    og_ref[...] = wg_ref[...].astype(jnp.bfloat16)
    ou_ref[...] = wu_ref[...].astype(jnp.bfloat16)
    od_ref[...] = wd_ref[...].astype(jnp.bfloat16)


def _add_kernel(a_ref, b_ref, o_ref):
    o_ref[...] = a_ref[...] + b_ref[...]


def _gmm_kernel(be_ref, xs_ref, ws_ref, wg_ref, wu_ref, wd_ref, o_ref):
    xs = xs_ref[...].astype(jnp.bfloat16)
    g = jnp.dot(xs, wg_ref[0], preferred_element_type=jnp.float32)
    u = jnp.dot(xs, wu_ref[0], preferred_element_type=jnp.float32)
    h = (g * lax.logistic(g)) * u * ws_ref[...]
    o_ref[...] = jnp.dot(h.astype(jnp.bfloat16), wd_ref[0],
                         preferred_element_type=jnp.float32)


def kernel(hidden_states, gate_w, w_gate, w_up, w_down,
           mlp_buffer=None, gathered_experts_out_buf=None):
    T, D = hidden_states.shape[0], hidden_states.shape[-1]
    E = gate_w.shape[0]
    F = w_gate.shape[-1]
    S_PAD = NBLK * BLK
    x = hidden_states.reshape(T, D)

    p0, p1, w0, w1, be = pl.pallas_call(
        _router_meta_kernel,
        in_specs=[
            pl.BlockSpec((T, D), lambda: (0, 0)),
            pl.BlockSpec((E, D), lambda: (0, 0)),
        ],
        out_specs=[
            pl.BlockSpec((T, 1), lambda: (0, 0)),
            pl.BlockSpec((T, 1), lambda: (0, 0)),
            pl.BlockSpec((T, 1), lambda: (0, 0)),
            pl.BlockSpec((T, 1), lambda: (0, 0)),
            pl.BlockSpec((1, NBLK), lambda: (0, 0)),
        ],
        out_shape=[
            jax.ShapeDtypeStruct((T, 1), jnp.int32),
            jax.ShapeDtypeStruct((T, 1), jnp.int32),
            jax.ShapeDtypeStruct((T, 1), jnp.float32),
            jax.ShapeDtypeStruct((T, 1), jnp.float32),
            jax.ShapeDtypeStruct((1, NBLK), jnp.int32),
        ],
    )(x, gate_w)

    wgb, wub, wdb = pl.pallas_call(
        _cast_kernel,
        grid=(E,),
        in_specs=[
            pl.BlockSpec((1, D, F), lambda e: (e, 0, 0)),
            pl.BlockSpec((1, D, F), lambda e: (e, 0, 0)),
            pl.BlockSpec((1, F, D), lambda e: (e, 0, 0)),
        ],
        out_specs=[
            pl.BlockSpec((1, D, F), lambda e: (e, 0, 0)),
            pl.BlockSpec((1, D, F), lambda e: (e, 0, 0)),
            pl.BlockSpec((1, F, D), lambda e: (e, 0, 0)),
        ],
        out_shape=[
            jax.ShapeDtypeStruct((E, D, F), jnp.bfloat16),
            jax.ShapeDtypeStruct((E, D, F), jnp.bfloat16),
            jax.ShapeDtypeStruct((E, F, D), jnp.bfloat16),
        ],
    )(w_gate, w_up, w_down)

    posk = jnp.concatenate([p0.reshape(T), p1.reshape(T)])
    tval = jnp.concatenate([jnp.arange(T, dtype=jnp.int32)] * 2)
    wk = jnp.concatenate([w0.reshape(T), w1.reshape(T)])
    n_per_w = S_PAD // NW
    n_chunk = n_per_w // 3

    mesh = plsc.VectorSubcoreMesh(core_axis_name="c", subcore_axis_name="s")
    sc_params = pltpu.CompilerParams()
    if "needs_layout_passes" in pltpu.CompilerParams.__dataclass_fields__:
        sc_params = dataclasses.replace(sc_params, needs_layout_passes=False)

    def _dispatch_body(x_hbm, posk_hbm, tval_hbm, wk_hbm, xs_hbm, ws_hbm,
                       pos_v, tok_v, wv, tsort_v, wsort_v, rows_v, sem):
        wid = lax.axis_index("s") * NC + lax.axis_index("c")
        pltpu.sync_copy(posk_hbm, pos_v)
        pltpu.sync_copy(tval_hbm, tok_v)
        pltpu.sync_copy(wk_hbm, wv)
        zi = jnp.zeros((16,), jnp.int32)
        zf = jnp.zeros((16,), jnp.float32)

        @pl.loop(0, S_PAD, step=16)
        def _(i):
            tsort_v.at[pl.ds(i, 16)][...] = zi
            wsort_v.at[pl.ds(i, 16)][...] = zf

        @pl.loop(0, 2 * T, step=16)
        def _(i):
            idxr = pos_v.at[pl.ds(i, 16)][...]
            plsc.store_scatter(tsort_v, [idxr], tok_v.at[pl.ds(i, 16)][...])
            plsc.store_scatter(wsort_v, [idxr], wv.at[pl.ds(i, 16)][...])

        base = wid * n_per_w
        for j in range(3):
            off = base + j * n_chunk
            pltpu.async_copy(x_hbm.at[tsort_v.at[pl.ds(off, n_chunk)]],
                             rows_v, sem).wait()
            pltpu.sync_copy(rows_v, xs_hbm.at[pl.ds(off, n_chunk)])

        @pl.when(wid == 0)
        def _():
            pltpu.sync_copy(wsort_v, ws_hbm)

    dispatch = pl.kernel(
        _dispatch_body,
        out_type=[
            jax.ShapeDtypeStruct((S_PAD, D), jnp.float32),
            jax.ShapeDtypeStruct((S_PAD,), jnp.float32),
        ],
        mesh=mesh,
        compiler_params=sc_params,
        scratch_types=[
            pltpu.VMEM((2 * T,), jnp.int32),
            pltpu.VMEM((2 * T,), jnp.int32),
            pltpu.VMEM((2 * T,), jnp.float32),
            pltpu.VMEM((S_PAD,), jnp.int32),
            pltpu.VMEM((S_PAD,), jnp.float32),
            pltpu.VMEM((n_chunk, D), jnp.float32),
            pltpu.SemaphoreType.DMA,
        ],
    )
    xs2, wsort = dispatch(x, posk, tval, wk)
    ws2 = wsort.reshape(S_PAD, 1)

    outs = pl.pallas_call(
        _gmm_kernel,
        grid_spec=pltpu.PrefetchScalarGridSpec(
            num_scalar_prefetch=1,
            grid=(NBLK,),
            in_specs=[
                pl.BlockSpec((BLK, D), lambda s, be: (s, 0)),
                pl.BlockSpec((BLK, 1), lambda s, be: (s, 0)),
                pl.BlockSpec((1, D, F), lambda s, be: (be[s], 0, 0)),
                pl.BlockSpec((1, D, F), lambda s, be: (be[s], 0, 0)),
                pl.BlockSpec((1, F, D), lambda s, be: (be[s], 0, 0)),
            ],
            out_specs=pl.BlockSpec((BLK, D), lambda s, be: (s, 0)),
        ),
        out_shape=jax.ShapeDtypeStruct((S_PAD, D), jnp.float32),
    )(be.reshape(NBLK), xs2, ws2, wgb, wub, wdb)

    t_per_w = T // NW
    t_half = t_per_w // 2

    def _combine_body(outs_hbm, posk_hbm, ab_hbm, idx_v, a_v, b_v, sem):
        wid = lax.axis_index("s") * NC + lax.axis_index("c")
        tbase = wid * t_per_w
        pltpu.sync_copy(posk_hbm.at[pl.ds(tbase, t_per_w)],
                        idx_v.at[pl.ds(0, t_per_w)])
        pltpu.sync_copy(posk_hbm.at[pl.ds(T + tbase, t_per_w)],
                        idx_v.at[pl.ds(t_per_w, t_per_w)])
        for c in range(2):
            pltpu.async_copy(
                outs_hbm.at[idx_v.at[pl.ds(c * t_half, t_half)]],
                a_v, sem).wait()
            pltpu.async_copy(
                outs_hbm.at[idx_v.at[pl.ds(t_per_w + c * t_half, t_half)]],
                b_v, sem).wait()
            pltpu.sync_copy(a_v, ab_hbm.at[pl.ds(tbase + c * t_half, t_half)])
            pltpu.sync_copy(b_v,
                            ab_hbm.at[pl.ds(T + tbase + c * t_half, t_half)])

    combine = pl.kernel(
        _combine_body,
        out_type=jax.ShapeDtypeStruct((2 * T, D), jnp.float32),
        mesh=mesh,
        compiler_params=sc_params,
        scratch_types=[
            pltpu.VMEM((2 * t_per_w,), jnp.int32),
            pltpu.VMEM((t_half, D), jnp.float32),
            pltpu.VMEM((t_half, D), jnp.float32),
            pltpu.SemaphoreType.DMA,
        ],
    )
    ab = combine(outs, posk)

    n_tb = 8
    tb = T // n_tb
    y = pl.pallas_call(
        _add_kernel,
        grid=(n_tb,),
        in_specs=[
            pl.BlockSpec((tb, D), lambda i: (i, 0)),
            pl.BlockSpec((tb, D), lambda i: (T // tb + i, 0)),
        ],
        out_specs=pl.BlockSpec((tb, D), lambda i: (i, 0)),
        out_shape=jax.ShapeDtypeStruct((T, D), jnp.float32),
    )(ab, ab)
    return y.reshape(hidden_states.shape)


# speedup vs baseline: 1.0094x; 1.0094x over previous
import dataclasses

import jax
import jax.numpy as jnp
from jax import lax
from jax.experimental import pallas as pl
from jax.experimental.pallas import tpu as pltpu
from jax.experimental.pallas import tpu_sc as plsc

K_TOP = 2
BLK = 256
NBLK = 24
NC, NS = 2, 16
NW = NC * NS


def _router_meta_kernel(x_ref, gw_ref, p0_ref, p1_ref,
                        w0_ref, w1_ref, be_ref):
    x = x_ref[...]
    T = x.shape[0]
    logits = jnp.dot(x, gw_ref[...].T,
                     preferred_element_type=jnp.float32)
    m = jnp.max(logits, axis=-1, keepdims=True)
    ex = jnp.exp(logits - m)
    p = ex / jnp.sum(ex, axis=-1, keepdims=True)
    num_experts = p.shape[-1]
    idx = lax.broadcasted_iota(jnp.int32, p.shape, 1)
    m1 = jnp.max(p, axis=-1, keepdims=True)
    i1 = jnp.min(jnp.where(p == m1, idx, num_experts), axis=-1, keepdims=True)
    is1 = idx == i1
    p2 = jnp.where(is1, -jnp.inf, p)
    m2 = jnp.max(p2, axis=-1, keepdims=True)
    i2 = jnp.min(jnp.where(p2 == m2, idx, num_experts), axis=-1, keepdims=True)
    is2 = idx == i2
    sel = is1 | is2
    wsum = m1 + m2
    w0_ref[...] = m1 / wsum
    w1_ref[...] = m2 / wsum

    selb = sel.astype(jnp.bfloat16)
    r_io = lax.broadcasted_iota(jnp.int32, (T, T), 0)
    c_io = lax.broadcasted_iota(jnp.int32, (T, T), 1)
    ltri = (c_io < r_io).astype(jnp.bfloat16)
    rank = jnp.dot(ltri, selb, preferred_element_type=jnp.float32)

    cnt = jnp.sum(sel.astype(jnp.float32), axis=0, keepdims=True)
    nblk = jnp.floor((cnt + (BLK - 1)) * (1.0 / BLK))
    e_r = lax.broadcasted_iota(jnp.int32, (num_experts, num_experts), 0)
    e_c = lax.broadcasted_iota(jnp.int32, (num_experts, num_experts), 1)
    excl = (e_r < e_c).astype(jnp.float32)
    blk_off = jnp.dot(nblk, excl,
                      preferred_element_type=jnp.float32)
    pad_off = blk_off * BLK
    pos = pad_off + rank
    p0_ref[...] = jnp.sum(jnp.where(is1, pos, 0.0), axis=1,
                          keepdims=True).astype(jnp.int32)
    p1_ref[...] = jnp.sum(jnp.where(is2, pos, 0.0), axis=1,
                          keepdims=True).astype(jnp.int32)

    blk_cum = blk_off + nblk
    lane8 = lax.broadcasted_iota(jnp.int32, (1, num_experts), 1)
    s_io = lax.broadcasted_iota(jnp.int32, (1, NBLK), 1).astype(jnp.float32)
    be = jnp.zeros((1, NBLK), jnp.int32)
    for e in range(num_experts):
        ce = jnp.sum(jnp.where(lane8 == e, blk_cum, 0.0),
                     axis=1, keepdims=True)
        be = be + (s_io >= ce).astype(jnp.int32)
    be_ref[...] = jnp.minimum(be, num_experts - 1)


def _cast_kernel(wg_ref, wu_ref, wd_ref, og_ref, ou_ref, od_ref):
    og_ref[...] = wg_ref[...].astype(jnp.bfloat16)
    ou_ref[...] = wu_ref[...].astype(jnp.bfloat16)
    od_ref[...] = wd_ref[...].astype(jnp.bfloat16)


def _add_kernel(a_ref, b_ref, o_ref):
    o_ref[...] = a_ref[...] + b_ref[...]


def _gmm_kernel(be_ref, xs_ref, ws_ref, wg_ref, wu_ref, wd_ref, o_ref):
    xs = xs_ref[...].astype(jnp.bfloat16)
    g = jnp.dot(xs, wg_ref[0], preferred_element_type=jnp.float32)
    u = jnp.dot(xs, wu_ref[0], preferred_element_type=jnp.float32)
    h = (g * lax.logistic(g)) * u * ws_ref[...]
    o_ref[...] = jnp.dot(h.astype(jnp.bfloat16), wd_ref[0],
                         preferred_element_type=jnp.float32)


def kernel(hidden_states, gate_w, w_gate, w_up, w_down,
           mlp_buffer=None, gathered_experts_out_buf=None):
    T, D = hidden_states.shape[0], hidden_states.shape[-1]
    E = gate_w.shape[0]
    F = w_gate.shape[-1]
    S_PAD = NBLK * BLK
    x = hidden_states.reshape(T, D)

    p0, p1, w0, w1, be = pl.pallas_call(
        _router_meta_kernel,
        in_specs=[
            pl.BlockSpec((T, D), lambda: (0, 0)),
            pl.BlockSpec((E, D), lambda: (0, 0)),
        ],
        out_specs=[
            pl.BlockSpec((T, 1), lambda: (0, 0)),
            pl.BlockSpec((T, 1), lambda: (0, 0)),
            pl.BlockSpec((T, 1), lambda: (0, 0)),
            pl.BlockSpec((T, 1), lambda: (0, 0)),
            pl.BlockSpec((1, NBLK), lambda: (0, 0)),
        ],
        out_shape=[
            jax.ShapeDtypeStruct((T, 1), jnp.int32),
            jax.ShapeDtypeStruct((T, 1), jnp.int32),
            jax.ShapeDtypeStruct((T, 1), jnp.float32),
            jax.ShapeDtypeStruct((T, 1), jnp.float32),
            jax.ShapeDtypeStruct((1, NBLK), jnp.int32),
        ],
    )(x, gate_w)

    wgb, wub, wdb = pl.pallas_call(
        _cast_kernel,
        grid=(E,),
        in_specs=[
            pl.BlockSpec((1, D, F), lambda e: (e, 0, 0)),
            pl.BlockSpec((1, D, F), lambda e: (e, 0, 0)),
            pl.BlockSpec((1, F, D), lambda e: (e, 0, 0)),
        ],
        out_specs=[
            pl.BlockSpec((1, D, F), lambda e: (e, 0, 0)),
            pl.BlockSpec((1, D, F), lambda e: (e, 0, 0)),
            pl.BlockSpec((1, F, D), lambda e: (e, 0, 0)),
        ],
        out_shape=[
            jax.ShapeDtypeStruct((E, D, F), jnp.bfloat16),
            jax.ShapeDtypeStruct((E, D, F), jnp.bfloat16),
            jax.ShapeDtypeStruct((E, F, D), jnp.bfloat16),
        ],
    )(w_gate, w_up, w_down)

    posk = jnp.concatenate([p0.reshape(T), p1.reshape(T)])
    tval = jnp.concatenate([jnp.arange(T, dtype=jnp.int32)] * 2)
    wk = jnp.concatenate([w0.reshape(T), w1.reshape(T)])
    n_per_w = S_PAD // NW
    n_chunk = n_per_w // 3

    mesh = plsc.VectorSubcoreMesh(core_axis_name="c", subcore_axis_name="s")
    sc_params = pltpu.CompilerParams()
    if "needs_layout_passes" in pltpu.CompilerParams.__dataclass_fields__:
        sc_params = dataclasses.replace(sc_params, needs_layout_passes=False)

    zeros_i = jnp.zeros((S_PAD,), jnp.int32)
    zeros_f = jnp.zeros((S_PAD,), jnp.float32)
    n_sub = (2 * T) // NS

    def _dispatch_body(x_hbm, posk_hbm, tval_hbm, wk_hbm, zi_hbm, zf_hbm,
                       xs_hbm, ws_hbm, pos_v, tok_v, wv, tloc_v, rows_v,
                       tsort_sh, wsort_sh, sem):
        cid = lax.axis_index("c")
        sid = lax.axis_index("s")
        wid = sid * NC + cid

        @pl.when(sid == 0)
        def _():
            pltpu.sync_copy(zi_hbm, tsort_sh)
            pltpu.sync_copy(zf_hbm, wsort_sh)

        sbase = sid * n_sub
        pltpu.sync_copy(posk_hbm.at[pl.ds(sbase, n_sub)], pos_v)
        pltpu.sync_copy(tval_hbm.at[pl.ds(sbase, n_sub)], tok_v)
        pltpu.sync_copy(wk_hbm.at[pl.ds(sbase, n_sub)], wv)
        plsc.subcore_barrier()
        for j in range(2):
            sl = pl.ds(j * (n_sub // 2), n_sub // 2)
            pltpu.sync_copy(tok_v.at[sl], tsort_sh.at[pos_v.at[sl]], add=True)
            pltpu.sync_copy(wv.at[sl], wsort_sh.at[pos_v.at[sl]], add=True)
        plsc.subcore_barrier()

        base = wid * n_per_w
        pltpu.sync_copy(tsort_sh.at[pl.ds(base, n_per_w)], tloc_v)
        for j in range(3):
            pltpu.async_copy(
                x_hbm.at[tloc_v.at[pl.ds(j * n_chunk, n_chunk)]],
                rows_v, sem).wait()
            pltpu.sync_copy(rows_v, xs_hbm.at[pl.ds(base + j * n_chunk,
                                                    n_chunk)])

        @pl.when(wid == 0)
        def _():
            pltpu.sync_copy(wsort_sh, ws_hbm)

    dispatch = pl.kernel(
        _dispatch_body,
        out_type=[
            jax.ShapeDtypeStruct((S_PAD, D), jnp.float32),
            jax.ShapeDtypeStruct((S_PAD,), jnp.float32),
        ],
        mesh=mesh,
        compiler_params=sc_params,
        scratch_types=[
            pltpu.VMEM((n_sub,), jnp.int32),
            pltpu.VMEM((n_sub,), jnp.int32),
            pltpu.VMEM((n_sub,), jnp.float32),
            pltpu.VMEM((n_per_w,), jnp.int32),
            pltpu.VMEM((n_chunk, D), jnp.float32),
            pltpu.VMEM_SHARED((S_PAD,), jnp.int32),
            pltpu.VMEM_SHARED((S_PAD,), jnp.float32),
            pltpu.SemaphoreType.DMA,
        ],
    )
    xs2, wsort = dispatch(x, posk, tval, wk, zeros_i, zeros_f)
    ws2 = wsort.reshape(S_PAD, 1)

    outs = pl.pallas_call(
        _gmm_kernel,
        grid_spec=pltpu.PrefetchScalarGridSpec(
            num_scalar_prefetch=1,
            grid=(NBLK,),
            in_specs=[
                pl.BlockSpec((BLK, D), lambda s, be: (s, 0)),
                pl.BlockSpec((BLK, 1), lambda s, be: (s, 0)),
                pl.BlockSpec((1, D, F), lambda s, be: (be[s], 0, 0)),
                pl.BlockSpec((1, D, F), lambda s, be: (be[s], 0, 0)),
                pl.BlockSpec((1, F, D), lambda s, be: (be[s], 0, 0)),
            ],
            out_specs=pl.BlockSpec((BLK, D), lambda s, be: (s, 0)),
        ),
        out_shape=jax.ShapeDtypeStruct((S_PAD, D), jnp.float32),
    )(be.reshape(NBLK), xs2, ws2, wgb, wub, wdb)

    t_per_w = T // NW
    t_half = t_per_w // 2

    def _combine_body(outs_hbm, posk_hbm, ab_hbm, idx_v, a_v, b_v, sem):
        wid = lax.axis_index("s") * NC + lax.axis_index("c")
        tbase = wid * t_per_w
        pltpu.sync_copy(posk_hbm.at[pl.ds(tbase, t_per_w)],
                        idx_v.at[pl.ds(0, t_per_w)])
        pltpu.sync_copy(posk_hbm.at[pl.ds(T + tbase, t_per_w)],
                        idx_v.at[pl.ds(t_per_w, t_per_w)])
        for c in range(2):
            pltpu.async_copy(
                outs_hbm.at[idx_v.at[pl.ds(c * t_half, t_half)]],
                a_v, sem).wait()
            pltpu.async_copy(
                outs_hbm.at[idx_v.at[pl.ds(t_per_w + c * t_half, t_half)]],
                b_v, sem).wait()
            pltpu.sync_copy(a_v, ab_hbm.at[pl.ds(tbase + c * t_half, t_half)])
            pltpu.sync_copy(b_v,
                            ab_hbm.at[pl.ds(T + tbase + c * t_half, t_half)])

    combine = pl.kernel(
        _combine_body,
        out_type=jax.ShapeDtypeStruct((2 * T, D), jnp.float32),
        mesh=mesh,
        compiler_params=sc_params,
        scratch_types=[
            pltpu.VMEM((2 * t_per_w,), jnp.int32),
            pltpu.VMEM((t_half, D), jnp.float32),
            pltpu.VMEM((t_half, D), jnp.float32),
            pltpu.SemaphoreType.DMA,
        ],
    )
    ab = combine(outs, posk)

    n_tb = 8
    tb = T // n_tb
    y = pl.pallas_call(
        _add_kernel,
        grid=(n_tb,),
        in_specs=[
            pl.BlockSpec((tb, D), lambda i: (i, 0)),
            pl.BlockSpec((tb, D), lambda i: (T // tb + i, 0)),
        ],
        out_specs=pl.BlockSpec((tb, D), lambda i: (i, 0)),
        out_shape=jax.ShapeDtypeStruct((T, D), jnp.float32),
    )(ab, ab)
    return y.reshape(hidden_states.shape)


# speedup vs baseline: 3.0714x; 3.0428x over previous
import jax
import jax.numpy as jnp
from jax.experimental import pallas as pl
from jax.experimental.pallas import tpu as pltpu

K_TOP = 2


def _moe_kernel(x_ref, xb_ref, gw_ref, wg_ref, wu_ref, wd_ref, o_ref, w_ref):
    e = pl.program_id(0)

    @pl.when(e == 0)
    def _router():
        x = x_ref[...]
        logits = jnp.dot(x, gw_ref[...].T,
                         preferred_element_type=jnp.float32)
        m = jnp.max(logits, axis=-1, keepdims=True)
        ex = jnp.exp(logits - m)
        p = ex / jnp.sum(ex, axis=-1, keepdims=True)
        num_experts = p.shape[-1]
        idx = jax.lax.broadcasted_iota(jnp.int32, p.shape, 1)
        m1 = jnp.max(p, axis=-1, keepdims=True)
        i1 = jnp.min(jnp.where(p == m1, idx, num_experts),
                     axis=-1, keepdims=True)
        is1 = idx == i1
        p2 = jnp.where(is1, -jnp.inf, p)
        m2 = jnp.max(p2, axis=-1, keepdims=True)
        i2 = jnp.min(jnp.where(p2 == m2, idx, num_experts),
                     axis=-1, keepdims=True)
        sel = is1 | (idx == i2)
        w = jnp.where(sel, p, 0.0)
        w_ref[...] = w / jnp.sum(w, axis=-1, keepdims=True)

    xb = xb_ref[...]
    w_all = w_ref[...]
    lane = jax.lax.broadcasted_iota(jnp.int32, w_all.shape, 1)
    we = jnp.sum(jnp.where(lane == e, w_all, 0.0),
                 axis=1, keepdims=True)
    wg = wg_ref[0].astype(jnp.bfloat16)
    wu = wu_ref[0].astype(jnp.bfloat16)
    wd = wd_ref[0].astype(jnp.bfloat16)
    g = jnp.dot(xb, wg, preferred_element_type=jnp.float32)
    u = jnp.dot(xb, wu, preferred_element_type=jnp.float32)
    h = (g * jax.lax.logistic(g)) * u
    hw = (h * we).astype(jnp.bfloat16)
    y = jnp.dot(hw, wd, preferred_element_type=jnp.float32)

    @pl.when(e == 0)
    def _init():
        o_ref[...] = y

    @pl.when(e != 0)
    def _acc():
        o_ref[...] += y


def kernel(hidden_states, gate_w, w_gate, w_up, w_down,
           mlp_buffer=None, gathered_experts_out_buf=None):
    T, D = hidden_states.shape[0], hidden_states.shape[-1]
    E = gate_w.shape[0]
    F = w_gate.shape[-1]
    x = hidden_states.reshape(T, D)
    xb = x.astype(jnp.bfloat16)

    out = pl.pallas_call(
        _moe_kernel,
        grid=(E,),
        in_specs=[
            pl.BlockSpec((T, D), lambda e: (0, 0)),
            pl.BlockSpec((T, D), lambda e: (0, 0)),
            pl.BlockSpec((E, D), lambda e: (0, 0)),
            pl.BlockSpec((1, D, F), lambda e: (e, 0, 0)),
            pl.BlockSpec((1, D, F), lambda e: (e, 0, 0)),
            pl.BlockSpec((1, F, D), lambda e: (e, 0, 0)),
        ],
        out_specs=pl.BlockSpec((T, D), lambda e: (0, 0)),
        out_shape=jax.ShapeDtypeStruct((T, D), jnp.float32),
        scratch_shapes=[pltpu.VMEM((T, E), jnp.float32)],
    )(x, xb, gate_w, w_gate, w_up, w_down)
    return out.reshape(hidden_states.shape)
